# Initial kernel scaffold; baseline (speedup 1.0000x reference)
import jax
import jax.numpy as jnp
from jax.experimental import pallas as pl


def kernel(x, edge_index, W1, b1, W2, b2):
    raise NotImplementedError("write your pallas kernel here")


# speedup vs baseline: 12.5197x; 12.5197x over previous
import functools

import jax
import jax.numpy as jnp
from jax import lax
from jax.experimental import pallas as pl
from jax.experimental.pallas import tpu as pltpu
from jax.experimental.pallas import tpu_sc as plsc

N = 10000
NP = 10240
TRASH = 10100
E = 320000
D = 128
H = 128
C = 40
CP = 48

NCORES = 2
NSUB = 16
NW = NCORES * NSUB
CHUNK = 128
NCHUNK = 80
EPW = CHUNK * NCHUNK
EPAD = NW * EPW

RPT = NP // NSUB
ZROWS = 64

@functools.cache
def _mesh():
  return plsc.VectorSubcoreMesh(
      core_axis_name="c", subcore_axis_name="s",
      num_cores=NCORES, num_subcores=NSUB)


def _zero_vmem_2d(buf, rows, width):
  per_row = width // 16

  @pl.loop(0, rows * per_row)
  def _(i):
    r = i // per_row
    col = (i % per_row) * 16
    buf[r, pl.ds(col, 16)] = jnp.zeros((16,), jnp.float32)


@functools.cache
def _make_sc_deg():
  return functools.partial(
      pl.kernel,
      out_type=jax.ShapeDtypeStruct((NCORES, NP), jnp.float32),
      mesh=_mesh(),
      scratch_types=[
          pltpu.VMEM_SHARED((NP,), jnp.float32),
          pltpu.VMEM((NCHUNK, CHUNK), jnp.int32),
          pltpu.VMEM((CHUNK,), jnp.float32),
          pltpu.VMEM((NP,), jnp.float32),
          pltpu.VMEM((RPT,), jnp.float32),
      ],
  )(_sc_deg_body)


def _sc_deg_body(dst_hbm, out_hbm, acc, idx_v, ones_v, stage_v, zbuf_v):
  cid = lax.axis_index("c")
  sid = lax.axis_index("s")
  w = cid * NSUB + sid

  @pl.loop(0, RPT // 16)
  def _(i):
    zbuf_v[pl.ds(i * 16, 16)] = jnp.zeros((16,), jnp.float32)

  @pl.loop(0, CHUNK // 16)
  def _(i):
    ones_v[pl.ds(i * 16, 16)] = jnp.ones((16,), jnp.float32)

  pltpu.sync_copy(dst_hbm.at[w], idx_v)
  pltpu.sync_copy(zbuf_v, acc.at[pl.ds(sid * RPT, RPT)])
  plsc.subcore_barrier()

  @pl.loop(0, NCHUNK)
  def _(c):
    pltpu.sync_copy(ones_v, acc.at[idx_v.at[c]], add=True)

  plsc.subcore_barrier()

  @pl.when(sid == 0)
  def _():
    pltpu.sync_copy(acc, stage_v)
    pltpu.sync_copy(stage_v, out_hbm.at[cid])


@functools.cache
def _make_sc_agg(width):
  @functools.partial(
      pl.kernel,
      out_type=jax.ShapeDtypeStruct((NCORES, NP, width), jnp.float32),
      mesh=_mesh(),
      scratch_types=[
          pltpu.VMEM_SHARED((NP, width), jnp.float32),
          pltpu.VMEM((NCHUNK, CHUNK), jnp.int32),
          pltpu.VMEM((NCHUNK, CHUNK), jnp.int32),
          pltpu.VMEM((CHUNK, width), jnp.float32),
          pltpu.VMEM((ZROWS, width), jnp.float32),
          pltpu.SemaphoreType.DMA,
      ],
      compiler_params=pltpu.CompilerParams(use_tc_tiling_on_sc=False),
  )
  def _sc_agg(g_hbm, src_hbm, dst_hbm, out_hbm, acc, sidx_v, didx_v, rowbuf,
              zbuf, gsem):
    cid = lax.axis_index("c")
    sid = lax.axis_index("s")
    w = cid * NSUB + sid

    _zero_vmem_2d(zbuf, ZROWS, width)
    pltpu.sync_copy(src_hbm.at[w], sidx_v)
    pltpu.sync_copy(dst_hbm.at[w], didx_v)

    @pl.loop(0, RPT // ZROWS)
    def _(j):
      pltpu.sync_copy(zbuf, acc.at[pl.ds(sid * RPT + j * ZROWS, ZROWS)])

    plsc.subcore_barrier()

    @pl.loop(0, NCHUNK)
    def _(c):
      pltpu.async_copy(g_hbm.at[sidx_v.at[c]], rowbuf, gsem).wait()
      pltpu.sync_copy(rowbuf, acc.at[didx_v.at[c]], add=True)

    plsc.subcore_barrier()

    @pl.loop(0, RPT // ZROWS)
    def _(j):
      start = sid * RPT + j * ZROWS
      pltpu.sync_copy(acc.at[pl.ds(start, ZROWS)], zbuf)
      pltpu.sync_copy(zbuf, out_hbm.at[cid, pl.ds(start, ZROWS)])

  return _sc_agg


BR = 512


def _dis(deg0_ref, deg1_ref):
  return lax.rsqrt(deg0_ref[...] + deg1_ref[...] + 1.0)


def _tc_g1_body(deg0_ref, deg1_ref, x_ref, w1_ref, g1_ref):
  h = jnp.dot(x_ref[...], w1_ref[...], preferred_element_type=jnp.float32)
  g1_ref[...] = h * _dis(deg0_ref, deg1_ref)


def _tc_mid_body(deg0_ref, deg1_ref, a0_ref, a1_ref, g1_ref, b1_ref, w2_ref,
                 g2_ref):
  dis = _dis(deg0_ref, deg1_ref)
  pre = dis * (a0_ref[...] + a1_ref[...] + g1_ref[...]) + b1_ref[...]
  h2 = jnp.maximum(pre, 0.0)
  g2_ref[...] = (
      jnp.dot(h2, w2_ref[...], preferred_element_type=jnp.float32) * dis)


def _tc_out_body(deg0_ref, deg1_ref, a0_ref, a1_ref, g2_ref, b2_ref, logp_ref,
                 feat_ref):
  dis = _dis(deg0_ref, deg1_ref)
  pre = dis * (a0_ref[...] + a1_ref[...] + g2_ref[...]) + b2_ref[...]
  feat_ref[...] = pre
  col = lax.broadcasted_iota(jnp.int32, pre.shape, 1)
  z = jnp.where(col < C, pre, -jnp.inf)
  m = jnp.max(z, axis=1, keepdims=True)
  s = jnp.sum(jnp.exp(z - m), axis=1, keepdims=True)
  logp_ref[...] = z - (jnp.log(s) + m)


def _row_spec(width):
  return pl.BlockSpec((BR, width), lambda i: (i, 0))


def _full_spec(shape):
  return pl.BlockSpec(shape, lambda i: (0,) * len(shape))


_GRID = (NP // BR,)

_tc_g1 = pl.pallas_call(
    _tc_g1_body,
    grid=_GRID,
    in_specs=[_row_spec(1), _row_spec(1), _row_spec(D), _full_spec((D, H))],
    out_specs=_row_spec(H),
    out_shape=jax.ShapeDtypeStruct((NP, H), jnp.float32),
)

_tc_mid = pl.pallas_call(
    _tc_mid_body,
    grid=_GRID,
    in_specs=[
        _row_spec(1), _row_spec(1), _row_spec(H), _row_spec(H), _row_spec(H),
        _full_spec((1, H)), _full_spec((H, CP)),
    ],
    out_specs=_row_spec(CP),
    out_shape=jax.ShapeDtypeStruct((NP, CP), jnp.float32),
)

_tc_out = pl.pallas_call(
    _tc_out_body,
    grid=_GRID,
    in_specs=[
        _row_spec(1), _row_spec(1), _row_spec(CP), _row_spec(CP),
        _row_spec(CP), _full_spec((1, CP)),
    ],
    out_specs=[_row_spec(CP), _row_spec(CP)],
    out_shape=[
        jax.ShapeDtypeStruct((NP, CP), jnp.float32),
        jax.ShapeDtypeStruct((NP, CP), jnp.float32),
    ],
)


@jax.jit
def kernel(x, edge_index, W1, b1, W2, b2):
  src = edge_index[0]
  dst = edge_index[1]
  pad = EPAD - E
  src_p = jnp.concatenate(
      [src, jnp.zeros((pad,), src.dtype)]).reshape(NW, NCHUNK, CHUNK)
  dst_p = jnp.concatenate(
      [dst, jnp.full((pad,), TRASH, dst.dtype)]).reshape(NW, NCHUNK, CHUNK)
  x_p = jnp.pad(x, ((0, NP - N), (0, 0)))
  w2_p = jnp.pad(W2, ((0, 0), (0, CP - C)))
  b1_r = b1.reshape(1, H)
  b2_r = jnp.pad(b2, (0, CP - C)).reshape(1, CP)

  deg_parts = _make_sc_deg()(dst_p)
  deg0 = deg_parts[0].reshape(NP, 1)
  deg1 = deg_parts[1].reshape(NP, 1)

  g1 = _tc_g1(deg0, deg1, x_p, W1)
  agg1 = _make_sc_agg(H)(g1, src_p, dst_p)
  g2 = _tc_mid(deg0, deg1, agg1[0], agg1[1], g1, b1_r, w2_p)
  agg2 = _make_sc_agg(CP)(g2, src_p, dst_p)
  logp, feat = _tc_out(deg0, deg1, agg2[0], agg2[1], g2, b2_r)
  return (logp[:N, :C], feat[:N, :C])


# speedup vs baseline: 13.0452x; 1.0420x over previous
import functools

import jax
import jax.numpy as jnp
from jax import lax
from jax.experimental import pallas as pl
from jax.experimental.pallas import tpu as pltpu
from jax.experimental.pallas import tpu_sc as plsc

N = 10000
NP = 10240
TRASH = 10100
E = 320000
D = 128
H = 128
C = 40
CP = 48

NCORES = 2
NSUB = 16
NW = NCORES * NSUB
CHUNK = 128
NCHUNK = 80
EPW = CHUNK * NCHUNK
EPAD = NW * EPW

RPT = NP // NSUB
ZROWS = 64

@functools.cache
def _mesh():
  return plsc.VectorSubcoreMesh(
      core_axis_name="c", subcore_axis_name="s",
      num_cores=NCORES, num_subcores=NSUB)


def _zero_vmem_2d(buf, rows, width):
  per_row = width // 16

  @pl.loop(0, rows * per_row)
  def _(i):
    r = i // per_row
    col = (i % per_row) * 16
    buf[r, pl.ds(col, 16)] = jnp.zeros((16,), jnp.float32)


@functools.cache
def _make_sc_deg():
  return functools.partial(
      pl.kernel,
      out_type=jax.ShapeDtypeStruct((NCORES, NP), jnp.float32),
      mesh=_mesh(),
      scratch_types=[
          pltpu.VMEM_SHARED((NP,), jnp.float32),
          pltpu.VMEM((NCHUNK, CHUNK), jnp.int32),
          pltpu.VMEM((CHUNK,), jnp.float32),
          pltpu.VMEM((NP,), jnp.float32),
          pltpu.VMEM((RPT,), jnp.float32),
      ],
  )(_sc_deg_body)


def _sc_deg_body(dst_hbm, out_hbm, acc, idx_v, ones_v, stage_v, zbuf_v):
  cid = lax.axis_index("c")
  sid = lax.axis_index("s")
  w = cid * NSUB + sid

  @pl.loop(0, RPT // 16)
  def _(i):
    zbuf_v[pl.ds(i * 16, 16)] = jnp.zeros((16,), jnp.float32)

  @pl.loop(0, CHUNK // 16)
  def _(i):
    ones_v[pl.ds(i * 16, 16)] = jnp.ones((16,), jnp.float32)

  pltpu.sync_copy(dst_hbm.at[w], idx_v)
  pltpu.sync_copy(zbuf_v, acc.at[pl.ds(sid * RPT, RPT)])
  plsc.subcore_barrier()

  @pl.loop(0, NCHUNK)
  def _(c):
    pltpu.sync_copy(ones_v, acc.at[idx_v.at[c]], add=True)

  plsc.subcore_barrier()

  @pl.when(sid == 0)
  def _():
    pltpu.sync_copy(acc, stage_v)
    pltpu.sync_copy(stage_v, out_hbm.at[cid])


@functools.cache
def _make_sc_agg(width):
  @functools.partial(
      pl.kernel,
      out_type=jax.ShapeDtypeStruct((NCORES, NP, width), jnp.float32),
      mesh=_mesh(),
      scratch_types=[
          pltpu.VMEM_SHARED((NP, width), jnp.float32),
          pltpu.VMEM((NCHUNK, CHUNK), jnp.int32),
          pltpu.VMEM((2, CHUNK), jnp.int32),
          pltpu.VMEM((2, CHUNK), jnp.int32),
          pltpu.VMEM((CHUNK, width), jnp.float32),
          pltpu.VMEM((CHUNK, width), jnp.float32),
          pltpu.SemaphoreType.DMA,
          pltpu.SemaphoreType.DMA,
          pltpu.SemaphoreType.DMA,
          pltpu.SemaphoreType.DMA,
      ],
      compiler_params=pltpu.CompilerParams(use_tc_tiling_on_sc=False),
  )
  def _sc_agg(g_hbm, pk_hbm, out_hbm, acc, pk_v, sidx_v, didx_v, rba, rbb,
              gsa, gsb, ssa, ssb):
    cid = lax.axis_index("c")
    sid = lax.axis_index("s")
    w = cid * NSUB + sid

    def unpack(c, slot):
      @pl.loop(0, CHUNK // 16)
      def _(k):
        v = pk_v[c, pl.ds(k * 16, 16)]
        sidx_v[slot, pl.ds(k * 16, 16)] = lax.shift_right_logical(v, 14)
        didx_v[slot, pl.ds(k * 16, 16)] = lax.bitwise_and(v, 16383)

    _zero_vmem_2d(rba, CHUNK, width)
    pltpu.sync_copy(pk_hbm.at[w], pk_v)

    @pl.loop(0, RPT // CHUNK)
    def _(j):
      pltpu.sync_copy(rba, acc.at[pl.ds(sid * RPT + j * CHUNK, CHUNK)])

    plsc.subcore_barrier()

    npair = NCHUNK // 2
    unpack(0, 0)
    pltpu.async_copy(g_hbm.at[sidx_v.at[0]], rba, gsa)

    @pl.loop(0, npair)
    def _(p):
      c0 = 2 * p
      pltpu.make_async_copy(g_hbm.at[sidx_v.at[0]], rba, gsa).wait()

      @pl.when(p > 0)
      def _():
        pltpu.make_async_copy(rbb, acc.at[didx_v.at[1]], ssb).wait()

      unpack(c0 + 1, 1)
      pltpu.async_copy(g_hbm.at[sidx_v.at[1]], rbb, gsb)
      pltpu.async_copy(rba, acc.at[didx_v.at[0]], ssa, add=True)
      pltpu.make_async_copy(g_hbm.at[sidx_v.at[1]], rbb, gsb).wait()
      pltpu.make_async_copy(rba, acc.at[didx_v.at[0]], ssa).wait()

      @pl.when(p + 1 < npair)
      def _():
        unpack(c0 + 2, 0)
        pltpu.async_copy(g_hbm.at[sidx_v.at[0]], rba, gsa)

      pltpu.async_copy(rbb, acc.at[didx_v.at[1]], ssb, add=True)

    pltpu.make_async_copy(rbb, acc.at[didx_v.at[1]], ssb).wait()
    plsc.subcore_barrier()

    @pl.loop(0, RPT // CHUNK)
    def _(j):
      start = sid * RPT + j * CHUNK
      pltpu.sync_copy(acc.at[pl.ds(start, CHUNK)], rba)
      pltpu.sync_copy(rba, out_hbm.at[cid, pl.ds(start, CHUNK)])

  return _sc_agg


BR = 512


def _dis(deg0_ref, deg1_ref):
  return lax.rsqrt(deg0_ref[...] + deg1_ref[...] + 1.0)


def _tc_g1_body(deg0_ref, deg1_ref, x_ref, w1_ref, g1_ref):
  h = jnp.dot(x_ref[...], w1_ref[...], preferred_element_type=jnp.float32)
  g1_ref[...] = h * _dis(deg0_ref, deg1_ref)


def _tc_mid_body(deg0_ref, deg1_ref, a0_ref, a1_ref, g1_ref, b1_ref, w2_ref,
                 g2_ref):
  dis = _dis(deg0_ref, deg1_ref)
  pre = dis * (a0_ref[...] + a1_ref[...] + g1_ref[...]) + b1_ref[...]
  h2 = jnp.maximum(pre, 0.0)
  g2_ref[...] = (
      jnp.dot(h2, w2_ref[...], preferred_element_type=jnp.float32) * dis)


def _tc_out_body(deg0_ref, deg1_ref, a0_ref, a1_ref, g2_ref, b2_ref, logp_ref,
                 feat_ref):
  dis = _dis(deg0_ref, deg1_ref)
  pre = dis * (a0_ref[...] + a1_ref[...] + g2_ref[...]) + b2_ref[...]
  feat_ref[...] = pre
  col = lax.broadcasted_iota(jnp.int32, pre.shape, 1)
  z = jnp.where(col < C, pre, -jnp.inf)
  m = jnp.max(z, axis=1, keepdims=True)
  s = jnp.sum(jnp.exp(z - m), axis=1, keepdims=True)
  logp_ref[...] = z - (jnp.log(s) + m)


def _row_spec(width):
  return pl.BlockSpec((BR, width), lambda i: (i, 0))


def _full_spec(shape):
  return pl.BlockSpec(shape, lambda i: (0,) * len(shape))


_GRID = (NP // BR,)

_tc_g1 = pl.pallas_call(
    _tc_g1_body,
    grid=_GRID,
    in_specs=[_row_spec(1), _row_spec(1), _row_spec(D), _full_spec((D, H))],
    out_specs=_row_spec(H),
    out_shape=jax.ShapeDtypeStruct((NP, H), jnp.float32),
)

_tc_mid = pl.pallas_call(
    _tc_mid_body,
    grid=_GRID,
    in_specs=[
        _row_spec(1), _row_spec(1), _row_spec(H), _row_spec(H), _row_spec(H),
        _full_spec((1, H)), _full_spec((H, CP)),
    ],
    out_specs=_row_spec(CP),
    out_shape=jax.ShapeDtypeStruct((NP, CP), jnp.float32),
)

_tc_out = pl.pallas_call(
    _tc_out_body,
    grid=_GRID,
    in_specs=[
        _row_spec(1), _row_spec(1), _row_spec(CP), _row_spec(CP),
        _row_spec(CP), _full_spec((1, CP)),
    ],
    out_specs=[_row_spec(CP), _row_spec(CP)],
    out_shape=[
        jax.ShapeDtypeStruct((NP, CP), jnp.float32),
        jax.ShapeDtypeStruct((NP, CP), jnp.float32),
    ],
)


@jax.jit
def kernel(x, edge_index, W1, b1, W2, b2):
  src = edge_index[0]
  dst = edge_index[1]
  pad = EPAD - E
  src_p = jnp.concatenate(
      [src, jnp.zeros((pad,), src.dtype)]).reshape(NW, NCHUNK, CHUNK)
  trash = (N + jnp.arange(pad, dtype=dst.dtype) % (NP - N)).astype(dst.dtype)
  dst_p = jnp.concatenate([dst, trash]).reshape(NW, NCHUNK, CHUNK)
  pk_p = jnp.left_shift(src_p, 14) | dst_p
  x_p = jnp.pad(x, ((0, NP - N), (0, 0)))
  w2_p = jnp.pad(W2, ((0, 0), (0, CP - C)))
  b1_r = b1.reshape(1, H)
  b2_r = jnp.pad(b2, (0, CP - C)).reshape(1, CP)

  deg_parts = _make_sc_deg()(dst_p)
  deg0 = deg_parts[0].reshape(NP, 1)
  deg1 = deg_parts[1].reshape(NP, 1)

  g1 = _tc_g1(deg0, deg1, x_p, W1)
  agg1 = _make_sc_agg(H)(g1, pk_p)
  g2 = _tc_mid(deg0, deg1, agg1[0], agg1[1], g1, b1_r, w2_p)
  agg2 = _make_sc_agg(CP)(g2, pk_p)
  logp, feat = _tc_out(deg0, deg1, agg2[0], agg2[1], g2, b2_r)
  return (logp[:N, :C], feat[:N, :C])


# speedup vs baseline: 30.7748x; 2.3591x over previous
import functools

import jax
import jax.numpy as jnp
from jax import lax
from jax.experimental import pallas as pl
from jax.experimental.pallas import tpu as pltpu
from jax.experimental.pallas import tpu_sc as plsc

N = 10000
NP = 10240
E = 320000
D = 128
H = 128
C = 40
CP = 48

NCORES = 2
NSUB = 16
NW = NCORES * NSUB
CHUNK = 128
NCHUNK = 80
EPW = E // NW
PADW = NCHUNK * CHUNK - EPW

RPT = NP // NSUB


@functools.cache
def _mesh():
  return plsc.VectorSubcoreMesh(
      core_axis_name="c", subcore_axis_name="s",
      num_cores=NCORES, num_subcores=NSUB)


def _zero_vmem_2d(buf, rows, width):
  per_row = width // 16

  @pl.loop(0, rows * per_row)
  def _(i):
    r = i // per_row
    col = (i % per_row) * 16
    buf[r, pl.ds(col, 16)] = jnp.zeros((16,), jnp.float32)


@functools.cache
def _make_sc_deg():
  @functools.partial(
      pl.kernel,
      out_type=(
          jax.ShapeDtypeStruct((NP,), jnp.float32),
          jax.ShapeDtypeStruct((NP,), jnp.float32),
          jax.ShapeDtypeStruct((NW, NCHUNK, CHUNK), jnp.int32),
      ),
      mesh=_mesh(),
      scratch_types=[
          pltpu.VMEM_SHARED((NP,), jnp.float32),
          pltpu.VMEM((EPW,), jnp.int32),
          pltpu.VMEM((EPW,), jnp.int32),
          pltpu.VMEM((NCHUNK, CHUNK), jnp.int32),
          pltpu.VMEM((NCHUNK, CHUNK), jnp.int32),
          pltpu.VMEM((CHUNK,), jnp.float32),
          pltpu.VMEM((NP,), jnp.float32),
          pltpu.VMEM((RPT,), jnp.float32),
      ],
      compiler_params=pltpu.CompilerParams(use_tc_tiling_on_sc=False),
  )
  def _sc_deg(ei_hbm, deg0_hbm, deg1_hbm, pk_hbm, acc, sv, dv, pk2, dd2,
              ones_v, stage_v, zbuf_v):
    cid = lax.axis_index("c")
    sid = lax.axis_index("s")
    w = cid * NSUB + sid

    @pl.loop(0, RPT // 16)
    def _(i):
      zbuf_v[pl.ds(i * 16, 16)] = jnp.zeros((16,), jnp.float32)

    @pl.loop(0, CHUNK // 16)
    def _(i):
      ones_v[pl.ds(i * 16, 16)] = jnp.ones((16,), jnp.float32)

    pltpu.sync_copy(ei_hbm.at[0, pl.ds(w * EPW, EPW)], sv)
    pltpu.sync_copy(ei_hbm.at[1, pl.ds(w * EPW, EPW)], dv)
    pltpu.sync_copy(zbuf_v, acc.at[pl.ds(sid * RPT, RPT)])

    @pl.loop(0, EPW // 16)
    def _(i):
      c = i // (CHUNK // 16)
      col = (i % (CHUNK // 16)) * 16
      s = sv[pl.ds(i * 16, 16)]
      d = dv[pl.ds(i * 16, 16)]
      pk2[c, pl.ds(col, 16)] = (s << 14) | d
      dd2[c, pl.ds(col, 16)] = d

    @pl.loop(EPW // 16, NCHUNK * CHUNK // 16)
    def _(i):
      c = i // (CHUNK // 16)
      col = (i % (CHUNK // 16)) * 16
      loc = (i * 16 - EPW) + lax.iota(jnp.int32, 16)
      spad = w * PADW + loc
      dpad = N + loc
      pk2[c, pl.ds(col, 16)] = (spad << 14) | dpad
      dd2[c, pl.ds(col, 16)] = dpad

    pltpu.sync_copy(pk2, pk_hbm.at[w])
    plsc.subcore_barrier()

    @pl.loop(0, NCHUNK)
    def _(c):
      pltpu.sync_copy(ones_v, acc.at[dd2.at[c]], add=True)

    plsc.subcore_barrier()

    @pl.when(sid == 0)
    def _():
      pltpu.sync_copy(acc, stage_v)

      @pl.when(cid == 0)
      def _():
        pltpu.sync_copy(stage_v, deg0_hbm)

      @pl.when(cid == 1)
      def _():
        pltpu.sync_copy(stage_v, deg1_hbm)

  return _sc_deg


@functools.cache
def _make_sc_agg(width):
  @functools.partial(
      pl.kernel,
      out_type=(
          jax.ShapeDtypeStruct((NP, width), jnp.float32),
          jax.ShapeDtypeStruct((NP, width), jnp.float32),
      ),
      mesh=_mesh(),
      scratch_types=[
          pltpu.VMEM_SHARED((NP, width), jnp.float32),
          pltpu.VMEM((NCHUNK, CHUNK), jnp.int32),
          pltpu.VMEM((2, CHUNK), jnp.int32),
          pltpu.VMEM((2, CHUNK), jnp.int32),
          pltpu.VMEM((CHUNK, width), jnp.float32),
          pltpu.VMEM((CHUNK, width), jnp.float32),
          pltpu.SemaphoreType.DMA,
          pltpu.SemaphoreType.DMA,
          pltpu.SemaphoreType.DMA,
          pltpu.SemaphoreType.DMA,
      ],
      compiler_params=pltpu.CompilerParams(use_tc_tiling_on_sc=False),
  )
  def _sc_agg(g_hbm, pk_hbm, out0_hbm, out1_hbm, acc, pk_v, sidx_v, didx_v,
              rba, rbb, gsa, gsb, ssa, ssb):
    cid = lax.axis_index("c")
    sid = lax.axis_index("s")
    w = cid * NSUB + sid

    def unpack(c, slot):
      @pl.loop(0, CHUNK // 16)
      def _(k):
        v = pk_v[c, pl.ds(k * 16, 16)]
        sidx_v[slot, pl.ds(k * 16, 16)] = lax.shift_right_logical(v, 14)
        didx_v[slot, pl.ds(k * 16, 16)] = lax.bitwise_and(v, 16383)

    _zero_vmem_2d(rba, CHUNK, width)
    pltpu.sync_copy(pk_hbm.at[w], pk_v)

    @pl.loop(0, RPT // CHUNK)
    def _(j):
      pltpu.sync_copy(rba, acc.at[pl.ds(sid * RPT + j * CHUNK, CHUNK)])

    plsc.subcore_barrier()

    npair = NCHUNK // 2
    unpack(0, 0)
    pltpu.async_copy(g_hbm.at[sidx_v.at[0]], rba, gsa)

    @pl.loop(0, npair)
    def _(p):
      c0 = 2 * p
      pltpu.make_async_copy(g_hbm.at[sidx_v.at[0]], rba, gsa).wait()

      @pl.when(p > 0)
      def _():
        pltpu.make_async_copy(rbb, acc.at[didx_v.at[1]], ssb).wait()

      unpack(c0 + 1, 1)
      pltpu.async_copy(g_hbm.at[sidx_v.at[1]], rbb, gsb)
      pltpu.async_copy(rba, acc.at[didx_v.at[0]], ssa, add=True)
      pltpu.make_async_copy(g_hbm.at[sidx_v.at[1]], rbb, gsb).wait()
      pltpu.make_async_copy(rba, acc.at[didx_v.at[0]], ssa).wait()

      @pl.when(p + 1 < npair)
      def _():
        unpack(c0 + 2, 0)
        pltpu.async_copy(g_hbm.at[sidx_v.at[0]], rba, gsa)

      pltpu.async_copy(rbb, acc.at[didx_v.at[1]], ssb, add=True)

    pltpu.make_async_copy(rbb, acc.at[didx_v.at[1]], ssb).wait()
    plsc.subcore_barrier()

    @pl.loop(0, RPT // CHUNK)
    def _(j):
      start = sid * RPT + j * CHUNK
      pltpu.sync_copy(acc.at[pl.ds(start, CHUNK)], rba)

      @pl.when(cid == 0)
      def _():
        pltpu.sync_copy(rba, out0_hbm.at[pl.ds(start, CHUNK)])

      @pl.when(cid == 1)
      def _():
        pltpu.sync_copy(rba, out1_hbm.at[pl.ds(start, CHUNK)])

  return _sc_agg


BR = 2000
_GRID = (N // BR,)


def _tc_g1_body(deg0_ref, deg1_ref, x_ref, w1_ref, g1_ref, dis_ref):
  dis = lax.rsqrt(deg0_ref[...] + deg1_ref[...] + 1.0)
  dis_ref[...] = dis
  h = jnp.dot(x_ref[...], w1_ref[...], preferred_element_type=jnp.float32)
  g1_ref[...] = h * dis


def _tc_mid_body(dis_ref, a0_ref, a1_ref, g1_ref, b1_ref, w2_ref, g2_ref):
  dis = dis_ref[...]
  pre = dis * (a0_ref[...] + a1_ref[...] + g1_ref[...]) + b1_ref[...]
  h2 = jnp.maximum(pre, 0.0)
  g2_ref[...] = (
      jnp.dot(h2, w2_ref[...], preferred_element_type=jnp.float32) * dis)


def _tc_out_body(dis_ref, a0_ref, a1_ref, g2_ref, b2_ref, logp_ref, feat_ref):
  pre = dis_ref[...] * (a0_ref[...] + a1_ref[...] + g2_ref[...]) + b2_ref[...]
  feat_ref[...] = pre[:, :C]
  col = lax.broadcasted_iota(jnp.int32, pre.shape, 1)
  z = jnp.where(col < C, pre, -jnp.inf)
  m = jnp.max(z, axis=1, keepdims=True)
  s = jnp.sum(jnp.exp(z - m), axis=1, keepdims=True)
  logp_ref[...] = (z - (jnp.log(s) + m))[:, :C]


def _row_spec(width):
  return pl.BlockSpec((BR, width), lambda i: (i, 0))


def _full_spec(shape):
  return pl.BlockSpec(shape, lambda i: (0,) * len(shape))


_tc_g1 = pl.pallas_call(
    _tc_g1_body,
    grid=_GRID,
    in_specs=[_row_spec(1), _row_spec(1), _row_spec(D), _full_spec((D, H))],
    out_specs=[_row_spec(H), _row_spec(1)],
    out_shape=[
        jax.ShapeDtypeStruct((N, H), jnp.float32),
        jax.ShapeDtypeStruct((N, 1), jnp.float32),
    ],
)

_tc_mid = pl.pallas_call(
    _tc_mid_body,
    grid=_GRID,
    in_specs=[
        _row_spec(1), _row_spec(H), _row_spec(H), _row_spec(H),
        _full_spec((1, H)), _full_spec((H, CP)),
    ],
    out_specs=_row_spec(CP),
    out_shape=jax.ShapeDtypeStruct((N, CP), jnp.float32),
)

_tc_out = pl.pallas_call(
    _tc_out_body,
    grid=_GRID,
    in_specs=[
        _row_spec(1), _row_spec(CP), _row_spec(CP), _row_spec(CP),
        _full_spec((1, CP)),
    ],
    out_specs=[_row_spec(C), _row_spec(C)],
    out_shape=[
        jax.ShapeDtypeStruct((N, C), jnp.float32),
        jax.ShapeDtypeStruct((N, C), jnp.float32),
    ],
)


@jax.jit
def kernel(x, edge_index, W1, b1, W2, b2):
  w2_p = jnp.pad(W2, ((0, 0), (0, CP - C)))
  b1_r = b1.reshape(1, H)
  b2_r = jnp.pad(b2, (0, CP - C)).reshape(1, CP)

  deg0, deg1, pk = _make_sc_deg()(edge_index)
  deg0 = deg0.reshape(NP, 1)
  deg1 = deg1.reshape(NP, 1)

  g1, dis = _tc_g1(deg0, deg1, x, W1)
  a10, a11 = _make_sc_agg(H)(g1, pk)
  g2 = _tc_mid(dis, a10, a11, g1, b1_r, w2_p)
  a20, a21 = _make_sc_agg(CP)(g2, pk)
  logp, feat = _tc_out(dis, a20, a21, g2, b2_r)
  return (logp, feat)


# speedup vs baseline: 31.2672x; 1.0160x over previous
import functools

import jax
import jax.numpy as jnp
from jax import lax
from jax.experimental import pallas as pl
from jax.experimental.pallas import tpu as pltpu
from jax.experimental.pallas import tpu_sc as plsc

N = 10000
NP = 10240
E = 320000
D = 128
H = 128
C = 40
CP = 48

NCORES = 2
NSUB = 16
NW = NCORES * NSUB
CHUNK = 128
NCHUNK = 80
EPW = E // NW
PADW = NCHUNK * CHUNK - EPW

RPT = NP // NSUB


@functools.cache
def _mesh():
  return plsc.VectorSubcoreMesh(
      core_axis_name="c", subcore_axis_name="s",
      num_cores=NCORES, num_subcores=NSUB)


def _zero_vmem_2d(buf, rows, width):
  per_row = width // 16

  @pl.loop(0, rows * per_row)
  def _(i):
    r = i // per_row
    col = (i % per_row) * 16
    buf[r, pl.ds(col, 16)] = jnp.zeros((16,), jnp.float32)


@functools.cache
def _make_sc_deg():
  @functools.partial(
      pl.kernel,
      out_type=(
          jax.ShapeDtypeStruct((NP,), jnp.float32),
          jax.ShapeDtypeStruct((NP,), jnp.float32),
          jax.ShapeDtypeStruct((NW, NCHUNK, CHUNK), jnp.int32),
      ),
      mesh=_mesh(),
      scratch_types=[
          pltpu.VMEM_SHARED((NP,), jnp.float32),
          pltpu.VMEM((EPW,), jnp.int32),
          pltpu.VMEM((EPW,), jnp.int32),
          pltpu.VMEM((NCHUNK, CHUNK), jnp.int32),
          pltpu.VMEM((NCHUNK, CHUNK), jnp.int32),
          pltpu.VMEM((CHUNK,), jnp.float32),
          pltpu.VMEM((NP,), jnp.float32),
          pltpu.VMEM((RPT,), jnp.float32),
          pltpu.SemaphoreType.DMA,
      ],
      compiler_params=pltpu.CompilerParams(use_tc_tiling_on_sc=False),
  )
  def _sc_deg(ei_hbm, deg0_hbm, deg1_hbm, pk_hbm, acc, sv, dv, pk2, dd2,
              ones_v, stage_v, zbuf_v, sem):
    cid = lax.axis_index("c")
    sid = lax.axis_index("s")
    w = cid * NSUB + sid

    @pl.loop(0, RPT // 16)
    def _(i):
      zbuf_v[pl.ds(i * 16, 16)] = jnp.zeros((16,), jnp.float32)

    @pl.loop(0, CHUNK // 16)
    def _(i):
      ones_v[pl.ds(i * 16, 16)] = jnp.ones((16,), jnp.float32)

    pltpu.sync_copy(ei_hbm.at[0, pl.ds(w * EPW, EPW)], sv)
    pltpu.sync_copy(ei_hbm.at[1, pl.ds(w * EPW, EPW)], dv)
    pltpu.sync_copy(zbuf_v, acc.at[pl.ds(sid * RPT, RPT)])

    @pl.loop(0, EPW // 16)
    def _(i):
      c = i // (CHUNK // 16)
      col = (i % (CHUNK // 16)) * 16
      s = sv[pl.ds(i * 16, 16)]
      d = dv[pl.ds(i * 16, 16)]
      pk2[c, pl.ds(col, 16)] = (s << 14) | d
      dd2[c, pl.ds(col, 16)] = d

    @pl.loop(EPW // 16, NCHUNK * CHUNK // 16)
    def _(i):
      c = i // (CHUNK // 16)
      col = (i % (CHUNK // 16)) * 16
      loc = (i * 16 - EPW) + lax.iota(jnp.int32, 16)
      spad = w * PADW + loc
      dpad = N + loc
      pk2[c, pl.ds(col, 16)] = (spad << 14) | dpad
      dd2[c, pl.ds(col, 16)] = dpad

    pltpu.sync_copy(pk2, pk_hbm.at[w])
    plsc.subcore_barrier()

    @pl.loop(0, NCHUNK)
    def _(c):
      pltpu.async_copy(ones_v, acc.at[dd2.at[c]], sem, add=True)

    @pl.loop(0, NCHUNK)
    def _(c):
      pltpu.make_async_copy(ones_v, acc.at[dd2.at[c]], sem).wait()

    plsc.subcore_barrier()

    @pl.when(sid == 0)
    def _():
      pltpu.sync_copy(acc, stage_v)

      @pl.when(cid == 0)
      def _():
        pltpu.sync_copy(stage_v, deg0_hbm)

      @pl.when(cid == 1)
      def _():
        pltpu.sync_copy(stage_v, deg1_hbm)

  return _sc_deg


@functools.cache
def _make_sc_agg(width):
  @functools.partial(
      pl.kernel,
      out_type=(
          jax.ShapeDtypeStruct((NP, width), jnp.float32),
          jax.ShapeDtypeStruct((NP, width), jnp.float32),
      ),
      mesh=_mesh(),
      scratch_types=[
          pltpu.VMEM_SHARED((NP, width), jnp.float32),
          pltpu.VMEM((NCHUNK, CHUNK), jnp.int32),
          pltpu.VMEM((2, CHUNK), jnp.int32),
          pltpu.VMEM((2, CHUNK), jnp.int32),
          pltpu.VMEM((CHUNK, width), jnp.float32),
          pltpu.VMEM((CHUNK, width), jnp.float32),
          pltpu.SemaphoreType.DMA,
          pltpu.SemaphoreType.DMA,
          pltpu.SemaphoreType.DMA,
          pltpu.SemaphoreType.DMA,
      ],
      compiler_params=pltpu.CompilerParams(use_tc_tiling_on_sc=False),
  )
  def _sc_agg(g_hbm, pk_hbm, out0_hbm, out1_hbm, acc, pk_v, sidx_v, didx_v,
              rba, rbb, gsa, gsb, ssa, ssb):
    cid = lax.axis_index("c")
    sid = lax.axis_index("s")
    w = cid * NSUB + sid

    def unpack(c, slot):
      @pl.loop(0, CHUNK // 16)
      def _(k):
        v = pk_v[c, pl.ds(k * 16, 16)]
        sidx_v[slot, pl.ds(k * 16, 16)] = lax.shift_right_logical(v, 14)
        didx_v[slot, pl.ds(k * 16, 16)] = lax.bitwise_and(v, 16383)

    _zero_vmem_2d(rba, CHUNK, width)
    pltpu.sync_copy(pk_hbm.at[w], pk_v)

    @pl.loop(0, RPT // CHUNK)
    def _(j):
      pltpu.sync_copy(rba, acc.at[pl.ds(sid * RPT + j * CHUNK, CHUNK)])

    plsc.subcore_barrier()

    npair = NCHUNK // 2
    unpack(0, 0)
    pltpu.async_copy(g_hbm.at[sidx_v.at[0]], rba, gsa)

    @pl.loop(0, npair)
    def _(p):
      c0 = 2 * p
      pltpu.make_async_copy(g_hbm.at[sidx_v.at[0]], rba, gsa).wait()

      @pl.when(p > 0)
      def _():
        pltpu.make_async_copy(rbb, acc.at[didx_v.at[1]], ssb).wait()

      unpack(c0 + 1, 1)
      pltpu.async_copy(g_hbm.at[sidx_v.at[1]], rbb, gsb)
      pltpu.async_copy(rba, acc.at[didx_v.at[0]], ssa, add=True)
      pltpu.make_async_copy(g_hbm.at[sidx_v.at[1]], rbb, gsb).wait()
      pltpu.make_async_copy(rba, acc.at[didx_v.at[0]], ssa).wait()

      @pl.when(p + 1 < npair)
      def _():
        unpack(c0 + 2, 0)
        pltpu.async_copy(g_hbm.at[sidx_v.at[0]], rba, gsa)

      pltpu.async_copy(rbb, acc.at[didx_v.at[1]], ssb, add=True)

    pltpu.make_async_copy(rbb, acc.at[didx_v.at[1]], ssb).wait()
    plsc.subcore_barrier()

    @pl.loop(0, RPT // CHUNK)
    def _(j):
      start = sid * RPT + j * CHUNK
      pltpu.sync_copy(acc.at[pl.ds(start, CHUNK)], rba)

      @pl.when(cid == 0)
      def _():
        pltpu.sync_copy(rba, out0_hbm.at[pl.ds(start, CHUNK)])

      @pl.when(cid == 1)
      def _():
        pltpu.sync_copy(rba, out1_hbm.at[pl.ds(start, CHUNK)])

  return _sc_agg


BR = 2000
_GRID = (N // BR,)


def _tc_g1_body(deg0_ref, deg1_ref, x_ref, w1_ref, g1_ref, dis_ref):
  dis = lax.rsqrt(deg0_ref[...] + deg1_ref[...] + 1.0)
  dis_ref[...] = dis
  h = jnp.dot(x_ref[...], w1_ref[...], preferred_element_type=jnp.float32)
  g1_ref[...] = h * dis


def _tc_mid_body(dis_ref, a0_ref, a1_ref, g1_ref, b1_ref, w2_ref, g2_ref):
  dis = dis_ref[...]
  pre = dis * (a0_ref[...] + a1_ref[...] + g1_ref[...]) + b1_ref[...]
  h2 = jnp.maximum(pre, 0.0)
  g2_ref[...] = (
      jnp.dot(h2, w2_ref[...], preferred_element_type=jnp.float32) * dis)


def _tc_out_body(dis_ref, a0_ref, a1_ref, g2_ref, b2_ref, logp_ref, feat_ref):
  pre = dis_ref[...] * (a0_ref[...] + a1_ref[...] + g2_ref[...]) + b2_ref[...]
  feat_ref[...] = pre[:, :C]
  col = lax.broadcasted_iota(jnp.int32, pre.shape, 1)
  z = jnp.where(col < C, pre, -jnp.inf)
  m = jnp.max(z, axis=1, keepdims=True)
  s = jnp.sum(jnp.exp(z - m), axis=1, keepdims=True)
  logp_ref[...] = (z - (jnp.log(s) + m))[:, :C]


def _row_spec(width):
  return pl.BlockSpec((BR, width), lambda i: (i, 0))


def _full_spec(shape):
  return pl.BlockSpec(shape, lambda i: (0,) * len(shape))


_tc_g1 = pl.pallas_call(
    _tc_g1_body,
    grid=_GRID,
    in_specs=[_row_spec(1), _row_spec(1), _row_spec(D), _full_spec((D, H))],
    out_specs=[_row_spec(H), _row_spec(1)],
    out_shape=[
        jax.ShapeDtypeStruct((N, H), jnp.float32),
        jax.ShapeDtypeStruct((N, 1), jnp.float32),
    ],
)

_tc_mid = pl.pallas_call(
    _tc_mid_body,
    grid=_GRID,
    in_specs=[
        _row_spec(1), _row_spec(H), _row_spec(H), _row_spec(H),
        _full_spec((1, H)), _full_spec((H, CP)),
    ],
    out_specs=_row_spec(CP),
    out_shape=jax.ShapeDtypeStruct((N, CP), jnp.float32),
)

_tc_out = pl.pallas_call(
    _tc_out_body,
    grid=_GRID,
    in_specs=[
        _row_spec(1), _row_spec(CP), _row_spec(CP), _row_spec(CP),
        _full_spec((1, CP)),
    ],
    out_specs=[_row_spec(C), _row_spec(C)],
    out_shape=[
        jax.ShapeDtypeStruct((N, C), jnp.float32),
        jax.ShapeDtypeStruct((N, C), jnp.float32),
    ],
)


@jax.jit
def kernel(x, edge_index, W1, b1, W2, b2):
  w2_p = jnp.pad(W2, ((0, 0), (0, CP - C)))
  b1_r = b1.reshape(1, H)
  b2_r = jnp.pad(b2, (0, CP - C)).reshape(1, CP)

  deg0, deg1, pk = _make_sc_deg()(edge_index)
  deg0 = deg0.reshape(NP, 1)
  deg1 = deg1.reshape(NP, 1)

  g1, dis = _tc_g1(deg0, deg1, x, W1)
  a10, a11 = _make_sc_agg(H)(g1, pk)
  g2 = _tc_mid(dis, a10, a11, g1, b1_r, w2_p)
  a20, a21 = _make_sc_agg(CP)(g2, pk)
  logp, feat = _tc_out(dis, a20, a21, g2, b2_r)
  return (logp, feat)
